# baseline (speedup 1.0000x reference)
import jax
import jax.numpy as jnp
from jax import lax
from jax.experimental import pallas as pl
from jax.experimental.pallas import tpu as pltpu
from jax.experimental.pallas import tpu_sc as plsc

N_CLASSES = 19
N_BINS = 10
H = 512
W = 512
BATCH = 8
ROWS = 512
RSTEPS = H // ROWS
NBLOCKS = BATCH * RSTEPS
SLICE_BLOCKS = (6, 1, 1)

NUM_TILES = 32
LANES = 16


def _dense_body(x_ref, t_ref, out_ref):
    t = t_ref[0]
    x0 = x_ref[0, 0]
    m = x0
    tv = x0
    s = jnp.exp(x0)
    for c in range(1, N_CLASSES):
        xc = x_ref[0, c]
        m = jnp.maximum(m, xc)
        tv = jnp.where(t == c, xc, tv)
        s = s + jnp.exp(xc)
    ps = jnp.exp(m) / s
    correct = (tv == m).astype(jnp.int32)
    b = jnp.minimum(lax.convert_element_type(ps * 10.0, jnp.int32), 9)
    key = b * 2 + correct
    packed = lax.bitcast_convert_type(
        (lax.bitcast_convert_type(ps, jnp.int32) & -32) | key, jnp.float32)
    out_ref[...] = packed


def _dense_call(output, target, start, nblk):
    def in_map(n, start=start):
        g = start + n
        return (g // RSTEPS, 0, g % RSTEPS, 0)

    def t_map(n, start=start):
        g = start + n
        return (g // RSTEPS, g % RSTEPS, 0)

    return pl.pallas_call(
        _dense_body,
        grid=(nblk,),
        in_specs=[
            pl.BlockSpec((1, N_CLASSES, ROWS, W), in_map),
            pl.BlockSpec((1, ROWS, W), t_map),
        ],
        out_specs=pl.BlockSpec((ROWS, W), lambda n: (n, 0)),
        out_shape=jax.ShapeDtypeStruct((nblk * ROWS, W), jnp.float32),
        compiler_params=pltpu.CompilerParams(
            dimension_semantics=("arbitrary",)),
    )(output, target)


def _make_hist_body(tile_rows):
    def _hist_body(packed_hbm, conf_out, cnt_out, buf, conf_acc, cnt_acc):
        nc = 2
        wid = lax.axis_index("s") * nc + lax.axis_index("c")
        lanes = lax.iota(jnp.int32, LANES)
        ones = jnp.full((LANES,), 1.0, jnp.float32)
        zero16 = jnp.zeros((LANES,), jnp.float32)
        for r in range(N_BINS):
            conf_acc[pl.ds(r * LANES, LANES)] = zero16
        for r in range(2 * N_BINS):
            cnt_acc[pl.ds(r * LANES, LANES)] = zero16

        pltpu.sync_copy(packed_hbm.at[pl.ds(wid * tile_rows, tile_rows)], buf)

        @plsc.parallel_loop(0, tile_rows * W // LANES, unroll=8)
        def vec_body(j):
            row = lax.shift_right_logical(j, 5)
            col = lax.shift_left(lax.bitwise_and(j, 31), 4)
            v = buf[row, pl.ds(col, LANES)]
            vi = plsc.bitcast(v, jnp.int32)
            key = lax.bitwise_and(vi, 31)
            b = lax.shift_right_logical(key, 1)
            conf_idx = lax.shift_left(b, 4) + lanes
            cnt_idx = lax.shift_left(key, 4) + lanes
            plsc.addupdate_scatter(conf_acc, [conf_idx], v)
            plsc.addupdate_scatter(cnt_acc, [cnt_idx], ones)

        pltpu.sync_copy(conf_acc, conf_out.at[wid])
        pltpu.sync_copy(cnt_acc, cnt_out.at[wid])

    return _hist_body


def _hist_call(packed, nblk):
    tile_rows = nblk * ROWS // NUM_TILES
    mesh = plsc.VectorSubcoreMesh(core_axis_name="c", subcore_axis_name="s",
                                  num_cores=2, num_subcores=16)
    f = pl.kernel(
        _make_hist_body(tile_rows),
        out_type=[
            jax.ShapeDtypeStruct((NUM_TILES, N_BINS * LANES), jnp.float32),
            jax.ShapeDtypeStruct((NUM_TILES, 2 * N_BINS * LANES), jnp.float32),
        ],
        mesh=mesh,
        scratch_types=[
            pltpu.VMEM((tile_rows, W), jnp.float32),
            pltpu.VMEM((N_BINS * LANES,), jnp.float32),
            pltpu.VMEM((2 * N_BINS * LANES,), jnp.float32),
        ],
        compiler_params=pltpu.CompilerParams(needs_layout_passes=False),
    )
    return f(packed)


def _final_body(*refs):
    nslices = len(SLICE_BLOCKS)
    conf_refs = refs[:nslices]
    cnt_refs = refs[nslices:2 * nslices]
    out_ref = refs[2 * nslices]
    conf_t = jnp.sum(conf_refs[0][...], axis=0)
    cnt_t = jnp.sum(cnt_refs[0][...], axis=0)
    for k in range(1, nslices):
        conf_t = conf_t + jnp.sum(conf_refs[k][...], axis=0)
        cnt_t = cnt_t + jnp.sum(cnt_refs[k][...], axis=0)
    ns = []
    accs = []
    confs = []
    for b in range(N_BINS):
        n0 = jnp.sum(cnt_t[2 * b * LANES:(2 * b + 1) * LANES])
        n1 = jnp.sum(cnt_t[(2 * b + 1) * LANES:(2 * b + 2) * LANES])
        ns.append(n0 + n1)
        accs.append(n1)
        confs.append(jnp.sum(conf_t[b * LANES:(b + 1) * LANES]))
    total = ns[0]
    for b in range(1, N_BINS):
        total = total + ns[b]
    ece = jnp.float32(0.0)
    for b in range(N_BINS):
        denom = ns[b] + 1e-13
        avg_acc = accs[b] / denom
        avg_conf = confs[b] / denom
        diff = jnp.abs(avg_acc - avg_conf)
        ece = ece + diff * diff * (ns[b] / total)
    out_ref[0, 0] = ece


def _final_call(confs, cnts):
    return pl.pallas_call(
        _final_body,
        out_specs=pl.BlockSpec(memory_space=pltpu.SMEM),
        out_shape=jax.ShapeDtypeStruct((1, 1), jnp.float32),
    )(*confs, *cnts)


def kernel(output, target):
    target = target.astype(jnp.int32)
    confs = []
    cnts = []
    start = 0
    for nblk in SLICE_BLOCKS:
        packed = _dense_call(output, target, start, nblk)
        conf, cnt = _hist_call(packed, nblk)
        confs.append(conf)
        cnts.append(cnt)
        start += nblk
    ece = _final_call(confs, cnts)
    return ece[0, 0]


# speedup vs baseline: 1.1077x; 1.1077x over previous
import jax
import jax.numpy as jnp
from jax import lax
from jax.experimental import pallas as pl
from jax.experimental.pallas import tpu as pltpu
from jax.experimental.pallas import tpu_sc as plsc

N_CLASSES = 19
N_BINS = 10
H = 512
W = 512
BATCH = 8
ROWS = 256
RSTEPS = H // ROWS
NBLOCKS = BATCH * RSTEPS
SLICE_BLOCKS = (12, 3)
LAST_BLOCKS = 1

NUM_TILES = 32
LANES = 16


def _dense_body(x_ref, t_ref, out_ref):
    t = t_ref[0]
    x0 = x_ref[0, 0]
    m = x0
    tv = x0
    s = jnp.exp(x0)
    for c in range(1, N_CLASSES):
        xc = x_ref[0, c]
        m = jnp.maximum(m, xc)
        tv = jnp.where(t == c, xc, tv)
        s = s + jnp.exp(xc)
    ps = jnp.exp(m) / s
    correct = (tv == m).astype(jnp.int32)
    b = jnp.minimum(lax.convert_element_type(ps * 10.0, jnp.int32), 9)
    key = b * 2 + correct
    packed = lax.bitcast_convert_type(
        (lax.bitcast_convert_type(ps, jnp.int32) & -32) | key, jnp.float32)
    out_ref[...] = packed


def _dense_call(output, target, start, nblk):
    def in_map(n, start=start):
        g = start + n
        return (g // RSTEPS, 0, g % RSTEPS, 0)

    def t_map(n, start=start):
        g = start + n
        return (g // RSTEPS, g % RSTEPS, 0)

    return pl.pallas_call(
        _dense_body,
        grid=(nblk,),
        in_specs=[
            pl.BlockSpec((1, N_CLASSES, ROWS, W), in_map),
            pl.BlockSpec((1, ROWS, W), t_map),
        ],
        out_specs=pl.BlockSpec((ROWS, W), lambda n: (n, 0)),
        out_shape=jax.ShapeDtypeStruct((nblk * ROWS, W), jnp.float32),
        compiler_params=pltpu.CompilerParams(
            dimension_semantics=("arbitrary",)),
    )(output, target)


def _make_hist_body(tile_rows):
    def _hist_body(packed_hbm, conf_out, cnt_out, buf, conf_acc, cnt_acc):
        nc = 2
        wid = lax.axis_index("s") * nc + lax.axis_index("c")
        lanes = lax.iota(jnp.int32, LANES)
        ones = jnp.full((LANES,), 1.0, jnp.float32)
        zero16 = jnp.zeros((LANES,), jnp.float32)
        for r in range(N_BINS):
            conf_acc[pl.ds(r * LANES, LANES)] = zero16
        for r in range(2 * N_BINS):
            cnt_acc[pl.ds(r * LANES, LANES)] = zero16

        pltpu.sync_copy(packed_hbm.at[pl.ds(wid * tile_rows, tile_rows)], buf)

        @plsc.parallel_loop(0, tile_rows * W // LANES, unroll=8)
        def vec_body(j):
            row = lax.shift_right_logical(j, 5)
            col = lax.shift_left(lax.bitwise_and(j, 31), 4)
            v = buf[row, pl.ds(col, LANES)]
            vi = plsc.bitcast(v, jnp.int32)
            key = lax.bitwise_and(vi, 31)
            b = lax.shift_right_logical(key, 1)
            conf_idx = lax.shift_left(b, 4) + lanes
            cnt_idx = lax.shift_left(key, 4) + lanes
            plsc.addupdate_scatter(conf_acc, [conf_idx], v)
            plsc.addupdate_scatter(cnt_acc, [cnt_idx], ones)

        pltpu.sync_copy(conf_acc, conf_out.at[wid])
        pltpu.sync_copy(cnt_acc, cnt_out.at[wid])

    return _hist_body


def _hist_call(packed, nblk):
    tile_rows = nblk * ROWS // NUM_TILES
    mesh = plsc.VectorSubcoreMesh(core_axis_name="c", subcore_axis_name="s",
                                  num_cores=2, num_subcores=16)
    f = pl.kernel(
        _make_hist_body(tile_rows),
        out_type=[
            jax.ShapeDtypeStruct((NUM_TILES, N_BINS * LANES), jnp.float32),
            jax.ShapeDtypeStruct((NUM_TILES, 2 * N_BINS * LANES), jnp.float32),
        ],
        mesh=mesh,
        scratch_types=[
            pltpu.VMEM((tile_rows, W), jnp.float32),
            pltpu.VMEM((N_BINS * LANES,), jnp.float32),
            pltpu.VMEM((2 * N_BINS * LANES,), jnp.float32),
        ],
        compiler_params=pltpu.CompilerParams(needs_layout_passes=False),
    )
    return f(packed)


def _final_body(*refs):
    nslices = len(SLICE_BLOCKS)
    conf_refs = refs[:nslices]
    cnt_refs = refs[nslices:2 * nslices]
    packed_ref = refs[2 * nslices]
    out_ref = refs[2 * nslices + 1]
    conf_t = jnp.sum(conf_refs[0][...], axis=0)
    cnt_t = jnp.sum(cnt_refs[0][...], axis=0)
    for k in range(1, nslices):
        conf_t = conf_t + jnp.sum(conf_refs[k][...], axis=0)
        cnt_t = cnt_t + jnp.sum(cnt_refs[k][...], axis=0)
    pk = packed_ref[...]
    vi = lax.bitcast_convert_type(pk, jnp.int32)
    key = vi & 31
    corr = (key & 1).astype(jnp.float32)
    bn = lax.shift_right_logical(key, 1)
    zf = jnp.zeros_like(pk)
    of = jnp.ones_like(pk)
    ns = []
    accs = []
    confs = []
    for b in range(N_BINS):
        n0 = jnp.sum(cnt_t[2 * b * LANES:(2 * b + 1) * LANES])
        n1 = jnp.sum(cnt_t[(2 * b + 1) * LANES:(2 * b + 2) * LANES])
        msk = bn == b
        ce = jnp.sum(jnp.where(msk, pk, zf))
        ne = jnp.sum(jnp.where(msk, of, zf))
        ae = jnp.sum(jnp.where(msk, corr, zf))
        ns.append(n0 + n1 + ne)
        accs.append(n1 + ae)
        confs.append(jnp.sum(conf_t[b * LANES:(b + 1) * LANES]) + ce)
    total = ns[0]
    for b in range(1, N_BINS):
        total = total + ns[b]
    ece = jnp.float32(0.0)
    for b in range(N_BINS):
        denom = ns[b] + 1e-13
        avg_acc = accs[b] / denom
        avg_conf = confs[b] / denom
        diff = jnp.abs(avg_acc - avg_conf)
        ece = ece + diff * diff * (ns[b] / total)
    out_ref[0, 0] = ece


def _final_call(confs, cnts, packed_last):
    return pl.pallas_call(
        _final_body,
        out_specs=pl.BlockSpec(memory_space=pltpu.SMEM),
        out_shape=jax.ShapeDtypeStruct((1, 1), jnp.float32),
    )(*confs, *cnts, packed_last)


def kernel(output, target):
    target = target.astype(jnp.int32)
    confs = []
    cnts = []
    start = 0
    for nblk in SLICE_BLOCKS:
        packed = _dense_call(output, target, start, nblk)
        conf, cnt = _hist_call(packed, nblk)
        confs.append(conf)
        cnts.append(cnt)
        start += nblk
    packed_last = _dense_call(output, target, start, LAST_BLOCKS)
    ece = _final_call(confs, cnts, packed_last)
    return ece[0, 0]
